# Initial kernel scaffold; baseline (speedup 1.0000x reference)
import jax
import jax.numpy as jnp
from jax.experimental import pallas as pl


def kernel(input, W1, b1, W1g, b1g, W2, b2):
    raise NotImplementedError("write your pallas kernel here")


# speedup vs baseline: 1.1670x; 1.1670x over previous
import jax
import jax.numpy as jnp
from jax.experimental import pallas as pl


_BM = 512


def _fused_router_kernel(x_ref, w1_ref, b1_ref, w1g_ref, b1g_ref, w2_ref,
                         b2_ref, o_ref):
    x = x_ref[...]
    dn = (((1,), (1,)), ((), ()))
    h = jax.lax.dot_general(x, w1_ref[...], dn,
                            preferred_element_type=jnp.float32) + b1_ref[...]
    g = jax.lax.dot_general(x, w1g_ref[...], dn,
                            preferred_element_type=jnp.float32) + b1g_ref[...]
    h = jnp.maximum(h * jax.nn.sigmoid(g), 0.0)
    logits = jax.lax.dot_general(h, w2_ref[...], dn,
                                 preferred_element_type=jnp.float32) + b2_ref[...]
    m = jnp.max(logits, axis=1, keepdims=True)
    e = jnp.exp(logits - m)
    o_ref[...] = e / jnp.sum(e, axis=1, keepdims=True)


def kernel(input, W1, b1, W1g, b1g, W2, b2):
    tokens, d_in = input.shape
    hidden = W1.shape[0]
    experts = W2.shape[0]
    grid = (tokens // _BM,)
    return pl.pallas_call(
        _fused_router_kernel,
        grid=grid,
        in_specs=[
            pl.BlockSpec((_BM, d_in), lambda i: (i, 0)),
            pl.BlockSpec((hidden, d_in), lambda i: (0, 0)),
            pl.BlockSpec((1, hidden), lambda i: (0, 0)),
            pl.BlockSpec((hidden, d_in), lambda i: (0, 0)),
            pl.BlockSpec((1, hidden), lambda i: (0, 0)),
            pl.BlockSpec((experts, hidden), lambda i: (0, 0)),
            pl.BlockSpec((1, experts), lambda i: (0, 0)),
        ],
        out_specs=pl.BlockSpec((_BM, experts), lambda i: (i, 0)),
        out_shape=jax.ShapeDtypeStruct((tokens, experts), jnp.float32),
    )(input, W1, b1.reshape(1, hidden), W1g, b1g.reshape(1, hidden),
      W2, b2.reshape(1, experts))
